# baseline (speedup 1.0000x reference)
import functools

import jax
import jax.numpy as jnp
from jax.experimental import pallas as pl
from jax.experimental.pallas import tpu as pltpu

_N = 65536
_D = 512
_K = 1024
_INV_T = 2.0


def _loss_kernel(feat_ref, cent_ref, out_ref, cnorm_ref, s_ref, acc_ref, *, blk):
    i = pl.program_id(0)

    @pl.when(i == 0)
    def _prep():
        c = cent_ref[...]
        norm = jnp.sqrt(jnp.sum(c * c, axis=1, keepdims=True))
        cn = c / jnp.maximum(norm, 1e-12)
        cnorm_ref[...] = cn.astype(jnp.bfloat16)
        g = jax.lax.dot_general(
            cn, cn, (((1,), (1,)), ((), ())),
            preferred_element_type=jnp.float32,
        )
        s_ref[...] = jnp.sum(jnp.exp(g * _INV_T), axis=0, keepdims=True)
        acc_ref[...] = jnp.zeros((1, 1), jnp.float32)

    f = feat_ref[...]
    cn = cnorm_ref[...]
    prod = jax.lax.dot_general(
        f, cn, (((1,), (1,)), ((), ())),
        preferred_element_type=jnp.float32,
    ) * _INV_T
    m = jnp.max(prod, axis=1, keepdims=True)
    iota = jax.lax.broadcasted_iota(jnp.int32, (blk, _K), 1)
    masked = jnp.where(prod == m, iota, _K)
    idx = jnp.min(masked, axis=1, keepdims=True)
    s_pick = jnp.sum(
        jnp.where(iota == idx, s_ref[...], 0.0), axis=1, keepdims=True
    )
    term = m - jnp.log(jnp.exp(m) + s_pick)
    acc_ref[...] += jnp.sum(term, axis=0, keepdims=True)

    @pl.when(i == pl.num_programs(0) - 1)
    def _fin():
        out_ref[...] = -acc_ref[...] / _N


@functools.partial(jax.jit, static_argnames=("blk",))
def _run(features, centroids, blk=1024):
    out = pl.pallas_call(
        functools.partial(_loss_kernel, blk=blk),
        grid=(_N // blk,),
        in_specs=[
            pl.BlockSpec((blk, _D), lambda i: (i, 0)),
            pl.BlockSpec((_K, _D), lambda i: (0, 0)),
        ],
        out_specs=pl.BlockSpec((1, 1), lambda i: (0, 0)),
        out_shape=jax.ShapeDtypeStruct((1, 1), jnp.float32),
        scratch_shapes=[
            pltpu.VMEM((_K, _D), jnp.bfloat16),
            pltpu.VMEM((1, _K), jnp.float32),
            pltpu.VMEM((1, 1), jnp.float32),
        ],
    )(features, centroids)
    return out[0, 0]


def kernel(features, centroids):
    return _run(features.astype(jnp.bfloat16), centroids)


# speedup vs baseline: 1.3798x; 1.3798x over previous
import functools

import jax
import jax.numpy as jnp
from jax.experimental import pallas as pl
from jax.experimental.pallas import tpu as pltpu

_N = 65536
_D = 512
_K = 1024
_INV_T = 2.0


def _loss_kernel(feat_ref, cent_ref, out_ref, cnorm_ref, s_ref, acc_ref, *, blk):
    i = pl.program_id(0)

    @pl.when(i == 0)
    def _prep():
        c = cent_ref[...]
        norm = jnp.sqrt(jnp.sum(c * c, axis=1, keepdims=True))
        cn = c / jnp.maximum(norm, 1e-12)
        cnorm_ref[...] = cn.astype(jnp.bfloat16)
        g = jax.lax.dot_general(
            cn, cn, (((1,), (1,)), ((), ())),
            preferred_element_type=jnp.float32,
        )
        s_ref[...] = jnp.sum(jnp.exp(g * _INV_T), axis=0, keepdims=True)
        acc_ref[...] = jnp.zeros((1, 1), jnp.float32)

    f = feat_ref[...].astype(jnp.bfloat16)
    cn = cnorm_ref[...]
    prod = jax.lax.dot_general(
        f, cn, (((1,), (1,)), ((), ())),
        preferred_element_type=jnp.float32,
    ) * _INV_T
    m = jnp.max(prod, axis=1, keepdims=True)
    iota = jax.lax.broadcasted_iota(jnp.int32, (blk, _K), 1)
    masked = jnp.where(prod == m, iota, _K)
    idx = jnp.min(masked, axis=1, keepdims=True)
    s_pick = jnp.sum(
        jnp.where(iota == idx, s_ref[...], 0.0), axis=1, keepdims=True
    )
    term = m - jnp.log(jnp.exp(m) + s_pick)
    acc_ref[...] += jnp.sum(term, axis=0, keepdims=True)

    @pl.when(i == pl.num_programs(0) - 1)
    def _fin():
        out_ref[...] = -acc_ref[...] / _N


@functools.partial(jax.jit, static_argnames=("blk",))
def _run(features, centroids, blk=1024):
    out = pl.pallas_call(
        functools.partial(_loss_kernel, blk=blk),
        grid=(_N // blk,),
        in_specs=[
            pl.BlockSpec((blk, _D), lambda i: (i, 0)),
            pl.BlockSpec((_K, _D), lambda i: (0, 0)),
        ],
        out_specs=pl.BlockSpec((1, 1), lambda i: (0, 0)),
        out_shape=jax.ShapeDtypeStruct((1, 1), jnp.float32),
        scratch_shapes=[
            pltpu.VMEM((_K, _D), jnp.bfloat16),
            pltpu.VMEM((1, _K), jnp.float32),
            pltpu.VMEM((1, 1), jnp.float32),
        ],
    )(features, centroids)
    return out[0, 0]


def kernel(features, centroids):
    return _run(features, centroids)


# speedup vs baseline: 2.0058x; 1.4537x over previous
import functools

import jax
import jax.numpy as jnp
from jax.experimental import pallas as pl
from jax.experimental.pallas import tpu as pltpu

_N = 65536
_D = 512
_K = 1024
_INV_T = 2.0


def _loss_kernel(feat_ref, cent_ref, out_ref, cnorm_ref, s_ref, acc_ref, *, blk):
    i = pl.program_id(0)

    @pl.when(i == 0)
    def _prep():
        c = cent_ref[...]
        norm = jnp.sqrt(jnp.sum(c * c, axis=1, keepdims=True))
        cn = c / jnp.maximum(norm, 1e-12)
        cnorm_ref[...] = cn.astype(jnp.bfloat16) * jnp.bfloat16(2.0)
        g = jax.lax.dot_general(
            cn, cn, (((1,), (1,)), ((), ())),
            preferred_element_type=jnp.float32,
        )
        s_ref[...] = jnp.sum(jnp.exp(g * _INV_T), axis=0, keepdims=True)
        acc_ref[...] = jnp.zeros((1, 1), jnp.float32)

    f = feat_ref[...].astype(jnp.bfloat16)
    cn = cnorm_ref[...]
    prod = jax.lax.dot_general(
        f, cn, (((1,), (1,)), ((), ())),
        preferred_element_type=jnp.float32,
    )
    m = jnp.max(prod, axis=1, keepdims=True)
    s_pick = jnp.min(
        jnp.where(prod == m, s_ref[...], jnp.inf), axis=1, keepdims=True
    )
    term = m - jnp.log(jnp.exp(m) + s_pick)
    acc_ref[...] += jnp.sum(term, axis=0, keepdims=True)

    @pl.when(i == pl.num_programs(0) - 1)
    def _fin():
        out_ref[...] = -acc_ref[...] / _N


@functools.partial(jax.jit, static_argnames=("blk",))
def _run(features, centroids, blk=1024):
    out = pl.pallas_call(
        functools.partial(_loss_kernel, blk=blk),
        grid=(_N // blk,),
        in_specs=[
            pl.BlockSpec((blk, _D), lambda i: (i, 0)),
            pl.BlockSpec((_K, _D), lambda i: (0, 0)),
        ],
        out_specs=pl.BlockSpec((1, 1), lambda i: (0, 0)),
        out_shape=jax.ShapeDtypeStruct((1, 1), jnp.float32),
        scratch_shapes=[
            pltpu.VMEM((_K, _D), jnp.bfloat16),
            pltpu.VMEM((1, _K), jnp.float32),
            pltpu.VMEM((1, 1), jnp.float32),
        ],
    )(features, centroids)
    return out[0, 0]


def kernel(features, centroids):
    return _run(features, centroids)


# speedup vs baseline: 2.1100x; 1.0520x over previous
import functools

import jax
import jax.numpy as jnp
from jax.experimental import pallas as pl
from jax.experimental.pallas import tpu as pltpu

_N = 65536
_D = 512
_K = 1024
_INV_T = 2.0


def _loss_kernel(feat_ref, cent_ref, out_ref, cnorm_ref, s_ref, acc_ref, *, blk):
    i = pl.program_id(0)

    @pl.when(i == 0)
    def _prep():
        c = cent_ref[...]
        norm = jnp.sqrt(jnp.sum(c * c, axis=1, keepdims=True))
        cn = c / jnp.maximum(norm, 1e-12)
        cnorm_ref[...] = cn.astype(jnp.bfloat16) * jnp.bfloat16(2.0)
        g = jax.lax.dot_general(
            cn, cn, (((1,), (1,)), ((), ())),
            preferred_element_type=jnp.float32,
        )
        s_ref[...] = jnp.sum(jnp.exp(g * _INV_T), axis=0, keepdims=True)
        acc_ref[...] = jnp.zeros((1, 1), jnp.float32)

    f = feat_ref[...].astype(jnp.bfloat16)
    cn = cnorm_ref[...]
    prod = jax.lax.dot_general(
        f, cn, (((1,), (1,)), ((), ())),
        preferred_element_type=jnp.float32,
    )
    m = jnp.max(prod, axis=1, keepdims=True)
    s_pick = jnp.min(
        jnp.where(prod == m, s_ref[...], jnp.inf), axis=1, keepdims=True
    )
    term = m - jnp.log(jnp.exp(m) + s_pick)
    acc_ref[...] += jnp.sum(term, axis=0, keepdims=True)

    @pl.when(i == pl.num_programs(0) - 1)
    def _fin():
        out_ref[...] = -acc_ref[...] / _N


@functools.partial(jax.jit, static_argnames=("blk",))
def _run(features, centroids, blk=2048):
    out = pl.pallas_call(
        functools.partial(_loss_kernel, blk=blk),
        grid=(_N // blk,),
        in_specs=[
            pl.BlockSpec((blk, _D), lambda i: (i, 0)),
            pl.BlockSpec((_K, _D), lambda i: (0, 0)),
        ],
        out_specs=pl.BlockSpec((1, 1), lambda i: (0, 0)),
        out_shape=jax.ShapeDtypeStruct((1, 1), jnp.float32),
        scratch_shapes=[
            pltpu.VMEM((_K, _D), jnp.bfloat16),
            pltpu.VMEM((1, _K), jnp.float32),
            pltpu.VMEM((1, 1), jnp.float32),
        ],
    )(features, centroids)
    return out[0, 0]


def kernel(features, centroids):
    return _run(features, centroids)


# speedup vs baseline: 2.1578x; 1.0226x over previous
import functools

import jax
import jax.numpy as jnp
from jax.experimental import pallas as pl
from jax.experimental.pallas import tpu as pltpu

_N = 65536
_D = 512
_K = 1024
_INV_T = 2.0


def _loss_kernel(feat_ref, cent_ref, out_ref, cnorm_ref, s_ref, acc_ref, *, blk):
    i = pl.program_id(0)

    @pl.when(i == 0)
    def _prep():
        c = cent_ref[...]
        norm = jnp.sqrt(jnp.sum(c * c, axis=1, keepdims=True))
        cn = c / jnp.maximum(norm, 1e-12)
        cnorm_ref[...] = cn.astype(jnp.bfloat16) * jnp.bfloat16(2.0)
        g = jax.lax.dot_general(
            cn, cn, (((1,), (1,)), ((), ())),
            preferred_element_type=jnp.float32,
        )
        s_ref[...] = jnp.sum(jnp.exp(g * _INV_T), axis=0, keepdims=True)
        acc_ref[...] = jnp.zeros((1, 1), jnp.float32)

    f = feat_ref[...].astype(jnp.bfloat16)
    cn = cnorm_ref[...]
    prod = jax.lax.dot_general(
        f, cn, (((1,), (1,)), ((), ())),
        preferred_element_type=jnp.float32,
    )
    m = jnp.max(prod, axis=1, keepdims=True)
    s_pick = jnp.min(
        jnp.where(prod == m, s_ref[...], jnp.inf), axis=1, keepdims=True
    )
    term = m - jnp.log(jnp.exp(m) + s_pick)
    acc_ref[...] += jnp.sum(term, axis=0, keepdims=True)

    @pl.when(i == pl.num_programs(0) - 1)
    def _fin():
        out_ref[...] = -acc_ref[...] / _N


@functools.partial(jax.jit, static_argnames=("blk",))
def _run(features, centroids, blk=4096):
    out = pl.pallas_call(
        functools.partial(_loss_kernel, blk=blk),
        grid=(_N // blk,),
        in_specs=[
            pl.BlockSpec((blk, _D), lambda i: (i, 0)),
            pl.BlockSpec((_K, _D), lambda i: (0, 0)),
        ],
        out_specs=pl.BlockSpec((1, 1), lambda i: (0, 0)),
        out_shape=jax.ShapeDtypeStruct((1, 1), jnp.float32),
        scratch_shapes=[
            pltpu.VMEM((_K, _D), jnp.bfloat16),
            pltpu.VMEM((1, _K), jnp.float32),
            pltpu.VMEM((1, 1), jnp.float32),
        ],
    )(features, centroids)
    return out[0, 0]


def kernel(features, centroids):
    return _run(features, centroids)


# speedup vs baseline: 2.3995x; 1.1120x over previous
import functools

import jax
import jax.numpy as jnp
from jax.experimental import pallas as pl
from jax.experimental.pallas import tpu as pltpu

_N = 65536
_D = 512
_K = 1024
_INV_T = 2.0


def _loss_kernel(feat_ref, cent_ref, out_ref, cnorm_ref, s_ref, acc_ref, *, blk):
    i = pl.program_id(0)

    @pl.when(i == 0)
    def _prep():
        c = cent_ref[...]
        norm = jnp.sqrt(jnp.sum(c * c, axis=1, keepdims=True))
        cn = c / jnp.maximum(norm, 1e-12)
        cnorm_ref[...] = cn.astype(jnp.bfloat16) * jnp.bfloat16(2.0)
        g = jax.lax.dot_general(
            cn, cn, (((1,), (1,)), ((), ())),
            preferred_element_type=jnp.float32,
        )
        s_ref[...] = jnp.sum(
            jnp.exp(g * _INV_T), axis=0, keepdims=True
        ).astype(jnp.bfloat16)
        acc_ref[...] = jnp.zeros((1, 1), jnp.float32)

    f = feat_ref[...].astype(jnp.bfloat16)
    cn = cnorm_ref[...]
    prod = jax.lax.dot_general(
        f, cn, (((1,), (1,)), ((), ())),
        preferred_element_type=jnp.float32,
    ).astype(jnp.bfloat16)
    m16 = jnp.max(prod, axis=1, keepdims=True)
    s_pick16 = jnp.min(
        jnp.where(prod == m16, s_ref[...], jnp.bfloat16(jnp.inf)),
        axis=1, keepdims=True,
    )
    m = m16.astype(jnp.float32)
    term = m - jnp.log(jnp.exp(m) + s_pick16.astype(jnp.float32))
    acc_ref[...] += jnp.sum(term, axis=0, keepdims=True)

    @pl.when(i == pl.num_programs(0) - 1)
    def _fin():
        out_ref[...] = -acc_ref[...] / _N


@functools.partial(jax.jit, static_argnames=("blk",))
def _run(features, centroids, blk=4096):
    out = pl.pallas_call(
        functools.partial(_loss_kernel, blk=blk),
        grid=(_N // blk,),
        in_specs=[
            pl.BlockSpec((blk, _D), lambda i: (i, 0)),
            pl.BlockSpec((_K, _D), lambda i: (0, 0)),
        ],
        out_specs=pl.BlockSpec((1, 1), lambda i: (0, 0)),
        out_shape=jax.ShapeDtypeStruct((1, 1), jnp.float32),
        scratch_shapes=[
            pltpu.VMEM((_K, _D), jnp.bfloat16),
            pltpu.VMEM((1, _K), jnp.bfloat16),
            pltpu.VMEM((1, 1), jnp.float32),
        ],
    )(features, centroids)
    return out[0, 0]


def kernel(features, centroids):
    return _run(features, centroids)


# speedup vs baseline: 2.6787x; 1.1163x over previous
import functools

import jax
import jax.numpy as jnp
from jax.experimental import pallas as pl
from jax.experimental.pallas import tpu as pltpu

_N = 65536
_D = 512
_K = 1024
_INV_T = 2.0


def _loss_kernel(feat_ref, cent_ref, out_ref, cnorm_ref, s_ref, acc_ref, *, blk):
    i = pl.program_id(0)

    @pl.when(i == 0)
    def _prep():
        c = cent_ref[...]
        norm = jnp.sqrt(jnp.sum(c * c, axis=1, keepdims=True))
        cn = c / jnp.maximum(norm, 1e-12)
        cnorm_ref[...] = cn.astype(jnp.bfloat16) * jnp.bfloat16(2.0)
        g = jax.lax.dot_general(
            cn, cn, (((1,), (1,)), ((), ())),
            preferred_element_type=jnp.float32,
        )
        s_ref[...] = jnp.sum(
            jnp.exp(g * _INV_T), axis=0, keepdims=True
        ).astype(jnp.bfloat16)
        acc_ref[...] = jnp.zeros((1, 1), jnp.float32)

    cn = cnorm_ref[...]
    s16 = s_ref[...]
    acc = jnp.zeros((1, 1), jnp.float32)
    rt = 1024
    for r in range(blk // rt):
        f_r = feat_ref[pl.ds(r * rt, rt), :].astype(jnp.bfloat16)
        prod = jax.lax.dot_general(
            f_r, cn, (((1,), (1,)), ((), ())),
            preferred_element_type=jnp.float32,
        ).astype(jnp.bfloat16)
        m16 = jnp.max(prod, axis=1, keepdims=True)
        s_pick16 = jnp.min(
            jnp.where(prod == m16, s16, jnp.bfloat16(jnp.inf)),
            axis=1, keepdims=True,
        )
        m = m16.astype(jnp.float32)
        term = m - jnp.log(jnp.exp(m) + s_pick16.astype(jnp.float32))
        acc = acc + jnp.sum(term, axis=0, keepdims=True)
    acc_ref[...] += acc

    @pl.when(i == pl.num_programs(0) - 1)
    def _fin():
        out_ref[...] = -acc_ref[...] / _N


@functools.partial(jax.jit, static_argnames=("blk",))
def _run(features, centroids, blk=4096):
    out = pl.pallas_call(
        functools.partial(_loss_kernel, blk=blk),
        grid=(_N // blk,),
        in_specs=[
            pl.BlockSpec((blk, _D), lambda i: (i, 0)),
            pl.BlockSpec((_K, _D), lambda i: (0, 0)),
        ],
        out_specs=pl.BlockSpec((1, 1), lambda i: (0, 0)),
        out_shape=jax.ShapeDtypeStruct((1, 1), jnp.float32),
        scratch_shapes=[
            pltpu.VMEM((_K, _D), jnp.bfloat16),
            pltpu.VMEM((1, _K), jnp.bfloat16),
            pltpu.VMEM((1, 1), jnp.float32),
        ],
    )(features, centroids)
    return out[0, 0]


def kernel(features, centroids):
    return _run(features, centroids)


# speedup vs baseline: 2.7106x; 1.0119x over previous
import functools

import jax
import jax.numpy as jnp
from jax.experimental import pallas as pl
from jax.experimental.pallas import tpu as pltpu

_N = 65536
_D = 512
_K = 1024
_INV_T = 2.0


def _loss_kernel(feat_ref, cent_ref, out_ref, cnorm_ref, s_ref, acc_ref, *, blk):
    i = pl.program_id(0)

    @pl.when(i == 0)
    def _prep():
        c = cent_ref[...]
        norm = jnp.sqrt(jnp.sum(c * c, axis=1, keepdims=True))
        cn = c / jnp.maximum(norm, 1e-12)
        cnorm_ref[...] = cn.astype(jnp.bfloat16) * jnp.bfloat16(2.0)
        g = jax.lax.dot_general(
            cn, cn, (((1,), (1,)), ((), ())),
            preferred_element_type=jnp.float32,
        )
        s_ref[...] = jnp.sum(
            jnp.exp(g * _INV_T), axis=0, keepdims=True
        ).astype(jnp.bfloat16)
        acc_ref[...] = jnp.zeros((1, 1), jnp.float32)

    cn = cnorm_ref[...]
    s16 = s_ref[...]
    acc = jnp.zeros((1, 1), jnp.float32)
    rt = 1024
    for r in range(blk // rt):
        f_r = feat_ref[pl.ds(r * rt, rt), :].astype(jnp.bfloat16)
        prod = jax.lax.dot_general(
            f_r, cn, (((1,), (1,)), ((), ())),
            preferred_element_type=jnp.float32,
        ).astype(jnp.bfloat16)
        m16 = jnp.max(prod, axis=1, keepdims=True)
        s_pick16 = jnp.min(
            jnp.where(prod == m16, s16, jnp.bfloat16(jnp.inf)),
            axis=1, keepdims=True,
        )
        m = m16.astype(jnp.float32)
        term = m - jnp.log(jnp.exp(m) + s_pick16.astype(jnp.float32))
        acc = acc + jnp.sum(term, axis=0, keepdims=True)
    acc_ref[...] += acc

    @pl.when(i == pl.num_programs(0) - 1)
    def _fin():
        out_ref[...] = -acc_ref[...] / _N


@functools.partial(jax.jit, static_argnames=("blk",))
def _run(features, centroids, blk=8192):
    out = pl.pallas_call(
        functools.partial(_loss_kernel, blk=blk),
        grid=(_N // blk,),
        in_specs=[
            pl.BlockSpec((blk, _D), lambda i: (i, 0)),
            pl.BlockSpec((_K, _D), lambda i: (0, 0)),
        ],
        out_specs=pl.BlockSpec((1, 1), lambda i: (0, 0)),
        out_shape=jax.ShapeDtypeStruct((1, 1), jnp.float32),
        scratch_shapes=[
            pltpu.VMEM((_K, _D), jnp.bfloat16),
            pltpu.VMEM((1, _K), jnp.bfloat16),
            pltpu.VMEM((1, 1), jnp.float32),
        ],
    )(features, centroids)
    return out[0, 0]


def kernel(features, centroids):
    return _run(features, centroids)


# speedup vs baseline: 2.7497x; 1.0144x over previous
import functools

import jax
import jax.numpy as jnp
from jax.experimental import pallas as pl
from jax.experimental.pallas import tpu as pltpu

_N = 65536
_D = 512
_K = 1024
_INV_T = 2.0


def _loss_kernel(feat_ref, cent_ref, out_ref, cnorm_ref, s_ref, acc_ref, *, blk):
    i = pl.program_id(0)

    @pl.when(i == 0)
    def _prep():
        c = cent_ref[...]
        norm = jnp.sqrt(jnp.sum(c * c, axis=1, keepdims=True))
        cn = c / jnp.maximum(norm, 1e-12)
        cnorm_ref[...] = cn.astype(jnp.bfloat16) * jnp.bfloat16(2.0)
        g = jax.lax.dot_general(
            cn, cn, (((1,), (1,)), ((), ())),
            preferred_element_type=jnp.float32,
        )
        s_ref[...] = jnp.sum(
            jnp.exp(g * _INV_T), axis=0, keepdims=True
        ).astype(jnp.bfloat16)
        acc_ref[...] = jnp.zeros((1, 1), jnp.float32)

    cn = cnorm_ref[...]
    s16 = s_ref[...]
    acc = jnp.zeros((1, 1), jnp.float32)
    rt = 512
    for r in range(blk // rt):
        f_r = feat_ref[pl.ds(r * rt, rt), :].astype(jnp.bfloat16)
        prod = jax.lax.dot_general(
            f_r, cn, (((1,), (1,)), ((), ())),
            preferred_element_type=jnp.float32,
        ).astype(jnp.bfloat16)
        m16 = jnp.max(prod, axis=1, keepdims=True)
        s_pick16 = jnp.min(
            jnp.where(prod == m16, s16, jnp.bfloat16(jnp.inf)),
            axis=1, keepdims=True,
        )
        m = m16.astype(jnp.float32)
        term = m - jnp.log(jnp.exp(m) + s_pick16.astype(jnp.float32))
        acc = acc + jnp.sum(term, axis=0, keepdims=True)
    acc_ref[...] += acc

    @pl.when(i == pl.num_programs(0) - 1)
    def _fin():
        out_ref[...] = -acc_ref[...] / _N


@functools.partial(jax.jit, static_argnames=("blk",))
def _run(features, centroids, blk=8192):
    out = pl.pallas_call(
        functools.partial(_loss_kernel, blk=blk),
        grid=(_N // blk,),
        in_specs=[
            pl.BlockSpec((blk, _D), lambda i: (i, 0)),
            pl.BlockSpec((_K, _D), lambda i: (0, 0)),
        ],
        out_specs=pl.BlockSpec((1, 1), lambda i: (0, 0)),
        out_shape=jax.ShapeDtypeStruct((1, 1), jnp.float32),
        scratch_shapes=[
            pltpu.VMEM((_K, _D), jnp.bfloat16),
            pltpu.VMEM((1, _K), jnp.bfloat16),
            pltpu.VMEM((1, 1), jnp.float32),
        ],
    )(features, centroids)
    return out[0, 0]


def kernel(features, centroids):
    return _run(features, centroids)


# speedup vs baseline: 2.7955x; 1.0167x over previous
import functools

import jax
import jax.numpy as jnp
from jax.experimental import pallas as pl
from jax.experimental.pallas import tpu as pltpu

_N = 65536
_D = 512
_K = 1024
_INV_T = 2.0


def _loss_kernel(feat_ref, cent_ref, out_ref, cnorm_ref, s_ref, acc_ref, *, blk):
    i = pl.program_id(0)

    @pl.when(i == 0)
    def _prep():
        c = cent_ref[...]
        norm = jnp.sqrt(jnp.sum(c * c, axis=1, keepdims=True))
        cn = c / jnp.maximum(norm, 1e-12)
        cnorm_ref[...] = cn.astype(jnp.bfloat16) * jnp.bfloat16(2.0)
        g = jax.lax.dot_general(
            cn, cn, (((1,), (1,)), ((), ())),
            preferred_element_type=jnp.float32,
        )
        s_ref[...] = jnp.sum(
            jnp.exp(g * _INV_T), axis=0, keepdims=True
        ).astype(jnp.bfloat16)
        acc_ref[...] = jnp.zeros((1, 1), jnp.float32)

    cn = cnorm_ref[...]
    s16 = s_ref[...]
    acc = jnp.zeros((1, 1), jnp.float32)
    rt = 256
    for r in range(blk // rt):
        f_r = feat_ref[pl.ds(r * rt, rt), :].astype(jnp.bfloat16)
        prod = jax.lax.dot_general(
            f_r, cn, (((1,), (1,)), ((), ())),
            preferred_element_type=jnp.float32,
        ).astype(jnp.bfloat16)
        m16 = jnp.max(prod, axis=1, keepdims=True)
        s_pick16 = jnp.min(
            jnp.where(prod == m16, s16, jnp.bfloat16(jnp.inf)),
            axis=1, keepdims=True,
        )
        m = m16.astype(jnp.float32)
        term = m - jnp.log(jnp.exp(m) + s_pick16.astype(jnp.float32))
        acc = acc + jnp.sum(term, axis=0, keepdims=True)
    acc_ref[...] += acc

    @pl.when(i == pl.num_programs(0) - 1)
    def _fin():
        out_ref[...] = -acc_ref[...] / _N


@functools.partial(jax.jit, static_argnames=("blk",))
def _run(features, centroids, blk=8192):
    out = pl.pallas_call(
        functools.partial(_loss_kernel, blk=blk),
        grid=(_N // blk,),
        in_specs=[
            pl.BlockSpec((blk, _D), lambda i: (i, 0)),
            pl.BlockSpec((_K, _D), lambda i: (0, 0)),
        ],
        out_specs=pl.BlockSpec((1, 1), lambda i: (0, 0)),
        out_shape=jax.ShapeDtypeStruct((1, 1), jnp.float32),
        scratch_shapes=[
            pltpu.VMEM((_K, _D), jnp.bfloat16),
            pltpu.VMEM((1, _K), jnp.bfloat16),
            pltpu.VMEM((1, 1), jnp.float32),
        ],
    )(features, centroids)
    return out[0, 0]


def kernel(features, centroids):
    return _run(features, centroids)
